# Initial kernel scaffold; baseline (speedup 1.0000x reference)
import jax
import jax.numpy as jnp
from jax.experimental import pallas as pl


def kernel(group_xyz, new_xyz):
    raise NotImplementedError("write your pallas kernel here")


# speedup vs baseline: 30.2509x; 30.2509x over previous
import jax
import jax.numpy as jnp
from jax import lax
from jax.experimental import pallas as pl
from jax.experimental.pallas import tpu as pltpu
from jax.experimental.pallas import tpu_sc as plsc

B, C, N, K = 16, 3, 4096, 64
R = B * N
NW = 32
ROWS_PER_W = R // NW
G = 256
NGROUPS = ROWS_PER_W // G
F = 30


def _rsqrt(v):
    i = plsc.bitcast(v, jnp.int32)
    i = jnp.full((16,), 0x5F3759DF, jnp.int32) - lax.shift_right_arithmetic(
        i, jnp.full((16,), 1, jnp.int32))
    y = plsc.bitcast(i, jnp.float32)
    half = jnp.full((16,), 0.5, jnp.float32)
    three_half = jnp.full((16,), 1.5, jnp.float32)
    for _ in range(3):
        y = y * (three_half - half * v * y * y)
    return y


def _sc_body(gx_hbm, nx_hbm, out_hbm, xb, yb, zb, cb, ob, accx, accy, accz,
             accn):
    cid = lax.axis_index("c")
    sid = lax.axis_index("s")
    wid = sid * 2 + cid
    b = wid // 2
    n0 = (wid % 2) * ROWS_PER_W
    row0 = wid * ROWS_PER_W

    iota = lax.iota(jnp.int32, 16)
    iota8 = iota * 8
    ones = jnp.full((16,), 1.0, jnp.float32)
    zeros = jnp.zeros((16,), jnp.float32)
    i4 = jnp.full((16,), 4, jnp.int32)
    i2 = jnp.full((16,), 2, jnp.int32)
    i1 = jnp.full((16,), 1, jnp.int32)
    i0 = jnp.zeros((16,), jnp.int32)

    def group_body(g, _):
        nrow = n0 + g * G
        grow = row0 + g * G
        pltpu.sync_copy(gx_hbm.at[pl.ds(((b * 3 + 0) * N + nrow) * K, G * K)],
                        xb)
        pltpu.sync_copy(gx_hbm.at[pl.ds(((b * 3 + 1) * N + nrow) * K, G * K)],
                        yb)
        pltpu.sync_copy(gx_hbm.at[pl.ds(((b * 3 + 2) * N + nrow) * K, G * K)],
                        zb)
        pltpu.sync_copy(nx_hbm.at[pl.ds(grow * 3, G * 3)], cb)

        def sub_body(s, _):
            r = s * 16 + iota
            rk = r * K
            r30 = r * F

            for ref in (accx, accy, accz, accn):
                for i in range(8):
                    ref[pl.ds(i * 16, 16)] = zeros

            sx = zeros
            sy = zeros
            sz = zeros
            sxx = zeros
            syy = zeros
            szz = zeros
            for k in range(K):
                idx = rk + jnp.full((16,), k, jnp.int32)
                x = plsc.load_gather(xb, [idx])
                y = plsc.load_gather(yb, [idx])
                z = plsc.load_gather(zb, [idx])
                oct_ = (lax.select(x > 0.0, i4, i0)
                        + lax.select(y > 0.0, i2, i0)
                        + lax.select(z > 0.0, i1, i0))
                a = iota8 + oct_
                plsc.addupdate_scatter(accx, [a], x)
                plsc.addupdate_scatter(accy, [a], y)
                plsc.addupdate_scatter(accz, [a], z)
                plsc.addupdate_scatter(accn, [a], ones)
                sx = sx + x
                sy = sy + y
                sz = sz + z
                sxx = sxx + x * x
                syy = syy + y * y
                szz = szz + z * z

            inv_k = jnp.full((16,), 1.0 / K, jnp.float32)
            inv_km1 = jnp.full((16,), 1.0 / (K - 1), jnp.float32)
            tiny = jnp.full((16,), 1e-30, jnp.float32)
            for c, (s1, s2) in enumerate(((sx, sxx), (sy, syy), (sz, szz))):
                var = (s2 - s1 * (s1 * inv_k)) * inv_km1
                var = jnp.maximum(var, zeros)
                std = var * _rsqrt(jnp.maximum(var, tiny))
                plsc.store_scatter(ob, [r30 + jnp.full((16,), c, jnp.int32)],
                                   std)

            r3 = r * 3
            for c in range(3):
                v = plsc.load_gather(cb, [r3 + jnp.full((16,), c, jnp.int32)])
                plsc.store_scatter(ob,
                                   [r30 + jnp.full((16,), 3 + c, jnp.int32)],
                                   v)

            for o in range(8):
                ao = iota8 + jnp.full((16,), o, jnp.int32)
                cnt = plsc.load_gather(accn, [ao])
                inv = ones / jnp.maximum(cnt, ones)
                for c, ref in enumerate((accx, accy, accz)):
                    m = plsc.load_gather(ref, [ao]) * inv
                    col = r30 + jnp.full((16,), 6 + o * 3 + c, jnp.int32)
                    plsc.store_scatter(ob, [col], m)
            return 0

        lax.fori_loop(0, G // 16, sub_body, 0)
        pltpu.sync_copy(ob, out_hbm.at[pl.ds(grow * F, G * F)])
        return 0

    lax.fori_loop(0, NGROUPS, group_body, 0)


@jax.jit
def kernel(group_xyz, new_xyz):
    gx = group_xyz.reshape(B * C * N * K)
    nx = new_xyz.reshape(R * 3)
    mesh = plsc.VectorSubcoreMesh(core_axis_name="c", subcore_axis_name="s")
    run = pl.kernel(
        _sc_body,
        out_type=jax.ShapeDtypeStruct((R * F,), jnp.float32),
        mesh=mesh,
        compiler_params=pltpu.CompilerParams(needs_layout_passes=False),
        scratch_types=[
            pltpu.VMEM((G * K,), jnp.float32),
            pltpu.VMEM((G * K,), jnp.float32),
            pltpu.VMEM((G * K,), jnp.float32),
            pltpu.VMEM((G * 3,), jnp.float32),
            pltpu.VMEM((G * F,), jnp.float32),
            pltpu.VMEM((128,), jnp.float32),
            pltpu.VMEM((128,), jnp.float32),
            pltpu.VMEM((128,), jnp.float32),
            pltpu.VMEM((128,), jnp.float32),
        ],
    )
    out = run(gx, nx)
    return out.reshape(B, N, F)


# speedup vs baseline: 47.2558x; 1.5621x over previous
import jax
import jax.numpy as jnp
from jax import lax
from jax.experimental import pallas as pl
from jax.experimental.pallas import tpu as pltpu
from jax.experimental.pallas import tpu_sc as plsc

B, C, N, K = 16, 3, 4096, 64
R = B * N
NW = 32
ROWS_PER_W = R // NW
G = 256
NGROUPS = ROWS_PER_W // G
F = 30


def _rsqrt(v):
    i = plsc.bitcast(v, jnp.int32)
    i = jnp.full((16,), 0x5F3759DF, jnp.int32) - lax.shift_right_arithmetic(
        i, jnp.full((16,), 1, jnp.int32))
    y = plsc.bitcast(i, jnp.float32)
    half = jnp.full((16,), 0.5, jnp.float32)
    three_half = jnp.full((16,), 1.5, jnp.float32)
    for _ in range(3):
        y = y * (three_half - half * v * y * y)
    return y


def _sc_body(gx_hbm, nx_hbm, out_hbm, xb, yb, zb, cb, ob, accx, accy, accz,
             accn):
    cid = lax.axis_index("c")
    sid = lax.axis_index("s")
    wid = sid * 2 + cid
    b = wid // 2
    n0 = (wid % 2) * ROWS_PER_W
    row0 = wid * ROWS_PER_W

    iota = lax.iota(jnp.int32, 16)
    ones = jnp.full((16,), 1.0, jnp.float32)
    zeros = jnp.zeros((16,), jnp.float32)
    abase = iota + jnp.full((16,), 112, jnp.int32)
    c25 = jnp.full((16,), 25, jnp.int32)
    c26 = jnp.full((16,), 26, jnp.int32)
    c27 = jnp.full((16,), 27, jnp.int32)
    m64 = jnp.full((16,), 64, jnp.int32)
    m32 = jnp.full((16,), 32, jnp.int32)
    m16 = jnp.full((16,), 16, jnp.int32)
    m63 = jnp.full((16,), 63, jnp.int32)

    def group_body(g, _):
        nrow = n0 + g * G
        grow = row0 + g * G
        pltpu.sync_copy(gx_hbm.at[pl.ds(((b * 3 + 0) * N + nrow) * K, G * K)],
                        xb)
        pltpu.sync_copy(gx_hbm.at[pl.ds(((b * 3 + 1) * N + nrow) * K, G * K)],
                        yb)
        pltpu.sync_copy(gx_hbm.at[pl.ds(((b * 3 + 2) * N + nrow) * K, G * K)],
                        zb)
        pltpu.sync_copy(nx_hbm.at[pl.ds(grow * 3, G * 3)], cb)

        def sub_body(s, _):
            r = s * 16 + iota
            rk = r * K
            r30 = r * F

            for ref in (accx, accy, accz, accn):
                for i in range(8):
                    ref[pl.ds(i * 16, 16)] = zeros

            sxx = zeros
            syy = zeros
            szz = zeros
            for k in range(K):
                kv = (jnp.full((16,), k, jnp.int32) + iota) & m63
                idx = rk + kv
                x = plsc.load_gather(xb, [idx])
                y = plsc.load_gather(yb, [idx])
                z = plsc.load_gather(zb, [idx])
                sx = lax.shift_right_logical(
                    plsc.bitcast(x, jnp.int32), c25) & m64
                sy = lax.shift_right_logical(
                    plsc.bitcast(y, jnp.int32), c26) & m32
                sz = lax.shift_right_logical(
                    plsc.bitcast(z, jnp.int32), c27) & m16
                a = abase - (sx + sy + sz)
                plsc.addupdate_scatter(accx, [a], x)
                plsc.addupdate_scatter(accy, [a], y)
                plsc.addupdate_scatter(accz, [a], z)
                plsc.addupdate_scatter(accn, [a], ones)
                sxx = sxx + x * x
                syy = syy + y * y
                szz = szz + z * z

            tot = [zeros, zeros, zeros]
            for o in range(8):
                cnt = accn[pl.ds(o * 16, 16)]
                inv = ones / jnp.maximum(cnt, ones)
                for c, ref in enumerate((accx, accy, accz)):
                    v = ref[pl.ds(o * 16, 16)]
                    tot[c] = tot[c] + v
                    col = r30 + jnp.full((16,), 6 + o * 3 + c, jnp.int32)
                    plsc.store_scatter(ob, [col], v * inv)
            sx, sy, sz = tot

            inv_k = jnp.full((16,), 1.0 / K, jnp.float32)
            inv_km1 = jnp.full((16,), 1.0 / (K - 1), jnp.float32)
            tiny = jnp.full((16,), 1e-30, jnp.float32)
            for c, (s1, s2) in enumerate(((sx, sxx), (sy, syy), (sz, szz))):
                var = (s2 - s1 * (s1 * inv_k)) * inv_km1
                var = jnp.maximum(var, zeros)
                std = var * _rsqrt(jnp.maximum(var, tiny))
                plsc.store_scatter(ob, [r30 + jnp.full((16,), c, jnp.int32)],
                                   std)

            r3 = r * 3
            for c in range(3):
                v = plsc.load_gather(cb, [r3 + jnp.full((16,), c, jnp.int32)])
                plsc.store_scatter(ob,
                                   [r30 + jnp.full((16,), 3 + c, jnp.int32)],
                                   v)
            return 0

        lax.fori_loop(0, G // 16, sub_body, 0)
        pltpu.sync_copy(ob, out_hbm.at[pl.ds(grow * F, G * F)])
        return 0

    lax.fori_loop(0, NGROUPS, group_body, 0)


@jax.jit
def kernel(group_xyz, new_xyz):
    gx = group_xyz.reshape(B * C * N * K)
    nx = new_xyz.reshape(R * 3)
    mesh = plsc.VectorSubcoreMesh(core_axis_name="c", subcore_axis_name="s")
    run = pl.kernel(
        _sc_body,
        out_type=jax.ShapeDtypeStruct((R * F,), jnp.float32),
        mesh=mesh,
        compiler_params=pltpu.CompilerParams(needs_layout_passes=False),
        scratch_types=[
            pltpu.VMEM((G * K,), jnp.float32),
            pltpu.VMEM((G * K,), jnp.float32),
            pltpu.VMEM((G * K,), jnp.float32),
            pltpu.VMEM((G * 3,), jnp.float32),
            pltpu.VMEM((G * F,), jnp.float32),
            pltpu.VMEM((128,), jnp.float32),
            pltpu.VMEM((128,), jnp.float32),
            pltpu.VMEM((128,), jnp.float32),
            pltpu.VMEM((128,), jnp.float32),
        ],
    )
    out = run(gx, nx)
    return out.reshape(B, N, F)


# speedup vs baseline: 49.4094x; 1.0456x over previous
import jax
import jax.numpy as jnp
from jax import lax
from jax.experimental import pallas as pl
from jax.experimental.pallas import tpu as pltpu
from jax.experimental.pallas import tpu_sc as plsc

B, C, N, K = 16, 3, 4096, 64
R = B * N
NW = 32
ROWS_PER_W = R // NW
G = 256
NGROUPS = ROWS_PER_W // G
F = 30


def _rsqrt(v):
    i = plsc.bitcast(v, jnp.int32)
    i = jnp.full((16,), 0x5F3759DF, jnp.int32) - lax.shift_right_arithmetic(
        i, jnp.full((16,), 1, jnp.int32))
    y = plsc.bitcast(i, jnp.float32)
    half = jnp.full((16,), 0.5, jnp.float32)
    three_half = jnp.full((16,), 1.5, jnp.float32)
    for _ in range(3):
        y = y * (three_half - half * v * y * y)
    return y


def _sc_body(gx_hbm, nx_hbm, out_hbm, xb, yb, zb, cb, ob, accx, accy, accz,
             accn):
    cid = lax.axis_index("c")
    sid = lax.axis_index("s")
    wid = sid * 2 + cid
    b = wid // 2
    n0 = (wid % 2) * ROWS_PER_W
    row0 = wid * ROWS_PER_W

    iota = lax.iota(jnp.int32, 16)
    ones = jnp.full((16,), 1.0, jnp.float32)
    zeros = jnp.zeros((16,), jnp.float32)
    abase = iota + jnp.full((16,), 112, jnp.int32)
    c25 = jnp.full((16,), 25, jnp.int32)
    c26 = jnp.full((16,), 26, jnp.int32)
    c27 = jnp.full((16,), 27, jnp.int32)
    m64 = jnp.full((16,), 64, jnp.int32)
    m32 = jnp.full((16,), 32, jnp.int32)
    m16 = jnp.full((16,), 16, jnp.int32)
    m63 = jnp.full((16,), 63, jnp.int32)
    i1 = jnp.full((16,), 1, jnp.int32)

    def group_body(g, _):
        nrow = n0 + g * G
        grow = row0 + g * G
        pltpu.sync_copy(gx_hbm.at[pl.ds(((b * 3 + 0) * N + nrow) * K, G * K)],
                        xb)
        pltpu.sync_copy(gx_hbm.at[pl.ds(((b * 3 + 1) * N + nrow) * K, G * K)],
                        yb)
        pltpu.sync_copy(gx_hbm.at[pl.ds(((b * 3 + 2) * N + nrow) * K, G * K)],
                        zb)
        pltpu.sync_copy(nx_hbm.at[pl.ds(grow * 3, G * 3)], cb)

        def sub_body(s, _):
            r = s * 16 + iota
            rk = r * K
            r30 = r * F

            for ref in (accx, accy, accz, accn):
                for i in range(8):
                    ref[pl.ds(i * 16, 16)] = zeros

            def point_step(kv, sxx, syy, szz):
                idx = rk + kv
                x = plsc.load_gather(xb, [idx])
                y = plsc.load_gather(yb, [idx])
                z = plsc.load_gather(zb, [idx])
                sx = lax.shift_right_logical(
                    plsc.bitcast(x, jnp.int32), c25) & m64
                sy = lax.shift_right_logical(
                    plsc.bitcast(y, jnp.int32), c26) & m32
                sz = lax.shift_right_logical(
                    plsc.bitcast(z, jnp.int32), c27) & m16
                a = abase - (sx + sy + sz)
                plsc.addupdate_scatter(accx, [a], x)
                plsc.addupdate_scatter(accy, [a], y)
                plsc.addupdate_scatter(accz, [a], z)
                plsc.addupdate_scatter(accn, [a], ones)
                return ((kv + i1) & m63, sxx + x * x, syy + y * y,
                        szz + z * z)

            UNROLL = 8
            def k_body(_, carry):
                kv, sxx, syy, szz = carry
                for _ in range(UNROLL):
                    kv, sxx, syy, szz = point_step(kv, sxx, syy, szz)
                return kv, sxx, syy, szz

            _, sxx, syy, szz = lax.fori_loop(
                0, K // UNROLL, k_body, (iota, zeros, zeros, zeros))

            tot = [zeros, zeros, zeros]
            for o in range(8):
                cnt = accn[pl.ds(o * 16, 16)]
                inv = ones / jnp.maximum(cnt, ones)
                for c, ref in enumerate((accx, accy, accz)):
                    v = ref[pl.ds(o * 16, 16)]
                    tot[c] = tot[c] + v
                    col = r30 + jnp.full((16,), 6 + o * 3 + c, jnp.int32)
                    plsc.store_scatter(ob, [col], v * inv)
            sx, sy, sz = tot

            inv_k = jnp.full((16,), 1.0 / K, jnp.float32)
            inv_km1 = jnp.full((16,), 1.0 / (K - 1), jnp.float32)
            tiny = jnp.full((16,), 1e-30, jnp.float32)
            for c, (s1, s2) in enumerate(((sx, sxx), (sy, syy), (sz, szz))):
                var = (s2 - s1 * (s1 * inv_k)) * inv_km1
                var = jnp.maximum(var, zeros)
                std = var * _rsqrt(jnp.maximum(var, tiny))
                plsc.store_scatter(ob, [r30 + jnp.full((16,), c, jnp.int32)],
                                   std)

            r3 = r * 3
            for c in range(3):
                v = plsc.load_gather(cb, [r3 + jnp.full((16,), c, jnp.int32)])
                plsc.store_scatter(ob,
                                   [r30 + jnp.full((16,), 3 + c, jnp.int32)],
                                   v)
            return 0

        lax.fori_loop(0, G // 16, sub_body, 0)
        pltpu.sync_copy(ob, out_hbm.at[pl.ds(grow * F, G * F)])
        return 0

    lax.fori_loop(0, NGROUPS, group_body, 0)


@jax.jit
def kernel(group_xyz, new_xyz):
    gx = group_xyz.reshape(B * C * N * K)
    nx = new_xyz.reshape(R * 3)
    mesh = plsc.VectorSubcoreMesh(core_axis_name="c", subcore_axis_name="s")
    run = pl.kernel(
        _sc_body,
        out_type=jax.ShapeDtypeStruct((R * F,), jnp.float32),
        mesh=mesh,
        compiler_params=pltpu.CompilerParams(needs_layout_passes=False),
        scratch_types=[
            pltpu.VMEM((G * K,), jnp.float32),
            pltpu.VMEM((G * K,), jnp.float32),
            pltpu.VMEM((G * K,), jnp.float32),
            pltpu.VMEM((G * 3,), jnp.float32),
            pltpu.VMEM((G * F,), jnp.float32),
            pltpu.VMEM((128,), jnp.float32),
            pltpu.VMEM((128,), jnp.float32),
            pltpu.VMEM((128,), jnp.float32),
            pltpu.VMEM((128,), jnp.float32),
        ],
    )
    out = run(gx, nx)
    return out.reshape(B, N, F)


# speedup vs baseline: 50.7862x; 1.0279x over previous
import jax
import jax.numpy as jnp
from jax import lax
from jax.experimental import pallas as pl
from jax.experimental.pallas import tpu as pltpu
from jax.experimental.pallas import tpu_sc as plsc

B, C, N, K = 16, 3, 4096, 64
R = B * N
NW = 32
ROWS_PER_W = R // NW
G = 128
NGROUPS = ROWS_PER_W // G
F = 30


def _rsqrt(v):
    i = plsc.bitcast(v, jnp.int32)
    i = jnp.full((16,), 0x5F3759DF, jnp.int32) - lax.shift_right_arithmetic(
        i, jnp.full((16,), 1, jnp.int32))
    y = plsc.bitcast(i, jnp.float32)
    half = jnp.full((16,), 0.5, jnp.float32)
    three_half = jnp.full((16,), 1.5, jnp.float32)
    for _ in range(3):
        y = y * (three_half - half * v * y * y)
    return y


def _sc_body(gx_hbm, nx_hbm, out_hbm, xb, yb, zb, cb, ob, accx, accy, accz,
             accn):
    cid = lax.axis_index("c")
    sid = lax.axis_index("s")
    wid = sid * 2 + cid
    b = wid // 2
    n0 = (wid % 2) * ROWS_PER_W
    iota = lax.iota(jnp.int32, 16)
    ones = jnp.full((16,), 1.0, jnp.float32)
    zeros = jnp.zeros((16,), jnp.float32)
    abase = iota + jnp.full((16,), 112, jnp.int32)
    c25 = jnp.full((16,), 25, jnp.int32)
    c26 = jnp.full((16,), 26, jnp.int32)
    c27 = jnp.full((16,), 27, jnp.int32)
    m64 = jnp.full((16,), 64, jnp.int32)
    m32 = jnp.full((16,), 32, jnp.int32)
    m16 = jnp.full((16,), 16, jnp.int32)
    m63 = jnp.full((16,), 63, jnp.int32)
    i1 = jnp.full((16,), 1, jnp.int32)

    def group_body(g, _):
        nrow = n0 + g * G
        pltpu.sync_copy(gx_hbm.at[b, 0, pl.ds(nrow, G), :], xb)
        pltpu.sync_copy(gx_hbm.at[b, 1, pl.ds(nrow, G), :], yb)
        pltpu.sync_copy(gx_hbm.at[b, 2, pl.ds(nrow, G), :], zb)
        pltpu.sync_copy(nx_hbm.at[b, pl.ds(nrow, G), :], cb)

        def sub_body(s, _):
            r = s * 16 + iota

            for ref in (accx, accy, accz, accn):
                for i in range(8):
                    ref[pl.ds(i * 16, 16)] = zeros

            def point_step(kv, sxx, syy, szz):
                x = plsc.load_gather(xb, [r, kv])
                y = plsc.load_gather(yb, [r, kv])
                z = plsc.load_gather(zb, [r, kv])
                sx = lax.shift_right_logical(
                    plsc.bitcast(x, jnp.int32), c25) & m64
                sy = lax.shift_right_logical(
                    plsc.bitcast(y, jnp.int32), c26) & m32
                sz = lax.shift_right_logical(
                    plsc.bitcast(z, jnp.int32), c27) & m16
                a = abase - (sx + sy + sz)
                plsc.addupdate_scatter(accx, [a], x)
                plsc.addupdate_scatter(accy, [a], y)
                plsc.addupdate_scatter(accz, [a], z)
                plsc.addupdate_scatter(accn, [a], ones)
                return ((kv + i1) & m63, sxx + x * x, syy + y * y,
                        szz + z * z)

            UNROLL = 8
            def k_body(_, carry):
                kv, sxx, syy, szz = carry
                for _ in range(UNROLL):
                    kv, sxx, syy, szz = point_step(kv, sxx, syy, szz)
                return kv, sxx, syy, szz

            _, sxx, syy, szz = lax.fori_loop(
                0, K // UNROLL, k_body, (iota, zeros, zeros, zeros))

            tot = [zeros, zeros, zeros]
            for o in range(8):
                cnt = accn[pl.ds(o * 16, 16)]
                inv = ones / jnp.maximum(cnt, ones)
                for c, ref in enumerate((accx, accy, accz)):
                    v = ref[pl.ds(o * 16, 16)]
                    tot[c] = tot[c] + v
                    col = jnp.full((16,), 6 + o * 3 + c, jnp.int32)
                    plsc.store_scatter(ob, [r, col], v * inv)
            sx, sy, sz = tot

            inv_k = jnp.full((16,), 1.0 / K, jnp.float32)
            inv_km1 = jnp.full((16,), 1.0 / (K - 1), jnp.float32)
            tiny = jnp.full((16,), 1e-30, jnp.float32)
            for c, (s1, s2) in enumerate(((sx, sxx), (sy, syy), (sz, szz))):
                var = (s2 - s1 * (s1 * inv_k)) * inv_km1
                var = jnp.maximum(var, zeros)
                std = var * _rsqrt(jnp.maximum(var, tiny))
                plsc.store_scatter(ob, [r, jnp.full((16,), c, jnp.int32)],
                                   std)

            for c in range(3):
                v = plsc.load_gather(cb, [r, jnp.full((16,), c, jnp.int32)])
                plsc.store_scatter(ob, [r, jnp.full((16,), 3 + c, jnp.int32)],
                                   v)
            return 0

        lax.fori_loop(0, G // 16, sub_body, 0)
        pltpu.sync_copy(ob, out_hbm.at[b, pl.ds(nrow, G), :])
        return 0

    lax.fori_loop(0, NGROUPS, group_body, 0)


@jax.jit
def kernel(group_xyz, new_xyz):
    mesh = plsc.VectorSubcoreMesh(core_axis_name="c", subcore_axis_name="s")
    run = pl.kernel(
        _sc_body,
        out_type=jax.ShapeDtypeStruct((B, N, F), jnp.float32),
        mesh=mesh,
        compiler_params=pltpu.CompilerParams(needs_layout_passes=False),
        scratch_types=[
            pltpu.VMEM((G, K), jnp.float32),
            pltpu.VMEM((G, K), jnp.float32),
            pltpu.VMEM((G, K), jnp.float32),
            pltpu.VMEM((G, 3), jnp.float32),
            pltpu.VMEM((G, F), jnp.float32),
            pltpu.VMEM((128,), jnp.float32),
            pltpu.VMEM((128,), jnp.float32),
            pltpu.VMEM((128,), jnp.float32),
            pltpu.VMEM((128,), jnp.float32),
        ],
    )
    return run(group_xyz, new_xyz)


# speedup vs baseline: 50.8449x; 1.0012x over previous
import jax
import jax.numpy as jnp
from jax import lax
from jax.experimental import pallas as pl
from jax.experimental.pallas import tpu as pltpu
from jax.experimental.pallas import tpu_sc as plsc

B, C, N, K = 16, 3, 4096, 64
R = B * N
NW = 32
ROWS_PER_W = R // NW
G = 128
NGROUPS = ROWS_PER_W // G
F = 30


def _rsqrt(v):
    i = plsc.bitcast(v, jnp.int32)
    i = jnp.full((16,), 0x5F3759DF, jnp.int32) - lax.shift_right_arithmetic(
        i, jnp.full((16,), 1, jnp.int32))
    y = plsc.bitcast(i, jnp.float32)
    half = jnp.full((16,), 0.5, jnp.float32)
    three_half = jnp.full((16,), 1.5, jnp.float32)
    for _ in range(3):
        y = y * (three_half - half * v * y * y)
    return y


def _sc_body(gx_hbm, nx_hbm, out_hbm, xb, yb, zb, cb, ob, accx, accy, accz,
             accn):
    cid = lax.axis_index("c")
    sid = lax.axis_index("s")
    wid = sid * 2 + cid
    b = wid // 2
    n0 = (wid % 2) * ROWS_PER_W
    iota = lax.iota(jnp.int32, 16)
    ones = jnp.full((16,), 1.0, jnp.float32)
    zeros = jnp.zeros((16,), jnp.float32)
    abase = iota + jnp.full((16,), 112, jnp.int32)
    c25 = jnp.full((16,), 25, jnp.int32)
    c26 = jnp.full((16,), 26, jnp.int32)
    c27 = jnp.full((16,), 27, jnp.int32)
    m64 = jnp.full((16,), 64, jnp.int32)
    m32 = jnp.full((16,), 32, jnp.int32)
    m16 = jnp.full((16,), 16, jnp.int32)
    m63 = jnp.full((16,), 63, jnp.int32)
    i1 = jnp.full((16,), 1, jnp.int32)

    def group_body(g, _):
        nrow = n0 + g * G
        pltpu.sync_copy(gx_hbm.at[b, 0, pl.ds(nrow, G), :], xb)
        pltpu.sync_copy(gx_hbm.at[b, 1, pl.ds(nrow, G), :], yb)
        pltpu.sync_copy(gx_hbm.at[b, 2, pl.ds(nrow, G), :], zb)
        pltpu.sync_copy(nx_hbm.at[b, pl.ds(nrow, G), :], cb)

        def sub_body(s, _):
            r = s * 16 + iota
            rb = r * jnp.full((16,), 128, jnp.int32)
            zi = jnp.zeros((16,), jnp.int32)

            for ref in (accx, accy, accz, accn):
                for i in range(8):
                    ref[pl.ds(i * 16, 16)] = zeros

            def point_step(kv, sxx, syy, szz):
                idx = rb + kv
                x = plsc.load_gather(xb, [zi, idx])
                y = plsc.load_gather(yb, [zi, idx])
                z = plsc.load_gather(zb, [zi, idx])
                sx = lax.shift_right_logical(
                    plsc.bitcast(x, jnp.int32), c25) & m64
                sy = lax.shift_right_logical(
                    plsc.bitcast(y, jnp.int32), c26) & m32
                sz = lax.shift_right_logical(
                    plsc.bitcast(z, jnp.int32), c27) & m16
                a = abase - (sx + sy + sz)
                plsc.addupdate_scatter(accx, [a], x)
                plsc.addupdate_scatter(accy, [a], y)
                plsc.addupdate_scatter(accz, [a], z)
                plsc.addupdate_scatter(accn, [a], ones)
                return ((kv + i1) & m63, sxx + x * x, syy + y * y,
                        szz + z * z)

            UNROLL = 8
            def k_body(_, carry):
                kv, sxx, syy, szz = carry
                for _ in range(UNROLL):
                    kv, sxx, syy, szz = point_step(kv, sxx, syy, szz)
                return kv, sxx, syy, szz

            _, sxx, syy, szz = lax.fori_loop(
                0, K // UNROLL, k_body, (iota, zeros, zeros, zeros))

            tot = [zeros, zeros, zeros]
            for o in range(8):
                cnt = accn[pl.ds(o * 16, 16)]
                inv = ones / jnp.maximum(cnt, ones)
                for c, ref in enumerate((accx, accy, accz)):
                    v = ref[pl.ds(o * 16, 16)]
                    tot[c] = tot[c] + v
                    col = rb + jnp.full((16,), 6 + o * 3 + c, jnp.int32)
                    plsc.store_scatter(ob, [zi, col], v * inv)
            sx, sy, sz = tot

            inv_k = jnp.full((16,), 1.0 / K, jnp.float32)
            inv_km1 = jnp.full((16,), 1.0 / (K - 1), jnp.float32)
            tiny = jnp.full((16,), 1e-30, jnp.float32)
            for c, (s1, s2) in enumerate(((sx, sxx), (sy, syy), (sz, szz))):
                var = (s2 - s1 * (s1 * inv_k)) * inv_km1
                var = jnp.maximum(var, zeros)
                std = var * _rsqrt(jnp.maximum(var, tiny))
                plsc.store_scatter(
                    ob, [zi, rb + jnp.full((16,), c, jnp.int32)], std)

            for c in range(3):
                v = plsc.load_gather(
                    cb, [zi, rb + jnp.full((16,), c, jnp.int32)])
                plsc.store_scatter(
                    ob, [zi, rb + jnp.full((16,), 3 + c, jnp.int32)], v)
            return 0

        lax.fori_loop(0, G // 16, sub_body, 0)
        pltpu.sync_copy(ob, out_hbm.at[b, pl.ds(nrow, G), :])
        return 0

    lax.fori_loop(0, NGROUPS, group_body, 0)


@jax.jit
def kernel(group_xyz, new_xyz):
    mesh = plsc.VectorSubcoreMesh(core_axis_name="c", subcore_axis_name="s")
    run = pl.kernel(
        _sc_body,
        out_type=jax.ShapeDtypeStruct((B, N, F), jnp.float32),
        mesh=mesh,
        compiler_params=pltpu.CompilerParams(needs_layout_passes=False),
        scratch_types=[
            pltpu.VMEM((G, K), jnp.float32),
            pltpu.VMEM((G, K), jnp.float32),
            pltpu.VMEM((G, K), jnp.float32),
            pltpu.VMEM((G, 3), jnp.float32),
            pltpu.VMEM((G, F), jnp.float32),
            pltpu.VMEM((128,), jnp.float32),
            pltpu.VMEM((128,), jnp.float32),
            pltpu.VMEM((128,), jnp.float32),
            pltpu.VMEM((128,), jnp.float32),
        ],
    )
    return run(group_xyz, new_xyz)


# speedup vs baseline: 54.9568x; 1.0809x over previous
import jax
import jax.numpy as jnp
from jax import lax
from jax.experimental import pallas as pl
from jax.experimental.pallas import tpu as pltpu
from jax.experimental.pallas import tpu_sc as plsc

B, C, N, K = 16, 3, 4096, 64
R = B * N
NW = 32
ROWS_PER_W = R // NW
G = 128
NGROUPS = ROWS_PER_W // G
F = 30


def _rsqrt(v):
    i = plsc.bitcast(v, jnp.int32)
    i = jnp.full((16,), 0x5F3759DF, jnp.int32) - lax.shift_right_arithmetic(
        i, jnp.full((16,), 1, jnp.int32))
    y = plsc.bitcast(i, jnp.float32)
    half = jnp.full((16,), 0.5, jnp.float32)
    three_half = jnp.full((16,), 1.5, jnp.float32)
    for _ in range(3):
        y = y * (three_half - half * v * y * y)
    return y


def _sc_body(gx_hbm, nx_hbm, out_hbm, xb, yb, zb, cb, ob, accx, accy, accz,
             accn, sem):
    cid = lax.axis_index("c")
    sid = lax.axis_index("s")
    wid = sid * 2 + cid
    b = wid // 2
    n0 = (wid % 2) * ROWS_PER_W
    iota = lax.iota(jnp.int32, 16)
    ones = jnp.full((16,), 1.0, jnp.float32)
    zeros = jnp.zeros((16,), jnp.float32)
    abase = iota + jnp.full((16,), 112, jnp.int32)
    c25 = jnp.full((16,), 25, jnp.int32)
    c26 = jnp.full((16,), 26, jnp.int32)
    c27 = jnp.full((16,), 27, jnp.int32)
    m64 = jnp.full((16,), 64, jnp.int32)
    m32 = jnp.full((16,), 32, jnp.int32)
    m16 = jnp.full((16,), 16, jnp.int32)
    m63 = jnp.full((16,), 63, jnp.int32)
    i1 = jnp.full((16,), 1, jnp.int32)

    def group_body(g, _):
        nrow = n0 + g * G
        h0 = pltpu.async_copy(gx_hbm.at[b, 0, pl.ds(nrow, G), :], xb, sem)
        h1 = pltpu.async_copy(gx_hbm.at[b, 1, pl.ds(nrow, G), :], yb, sem)
        h2 = pltpu.async_copy(gx_hbm.at[b, 2, pl.ds(nrow, G), :], zb, sem)
        h3 = pltpu.async_copy(nx_hbm.at[b, pl.ds(nrow, G), :], cb, sem)
        h0.wait()
        h1.wait()
        h2.wait()
        h3.wait()

        def sub_body(s, _):
            r = s * 16 + iota
            rb = r * jnp.full((16,), 128, jnp.int32)
            zi = jnp.zeros((16,), jnp.int32)

            for ref in (accx, accy, accz, accn):
                for i in range(8):
                    ref[pl.ds(i * 16, 16)] = zeros

            def point_step(kv, sxx, syy, szz):
                idx = rb + kv
                x = plsc.load_gather(xb, [zi, idx])
                y = plsc.load_gather(yb, [zi, idx])
                z = plsc.load_gather(zb, [zi, idx])
                sx = lax.shift_right_logical(
                    plsc.bitcast(x, jnp.int32), c25) & m64
                sy = lax.shift_right_logical(
                    plsc.bitcast(y, jnp.int32), c26) & m32
                sz = lax.shift_right_logical(
                    plsc.bitcast(z, jnp.int32), c27) & m16
                a = abase - (sx + sy + sz)
                plsc.addupdate_scatter(accx, [a], x)
                plsc.addupdate_scatter(accy, [a], y)
                plsc.addupdate_scatter(accz, [a], z)
                plsc.addupdate_scatter(accn, [a], ones)
                return ((kv + i1) & m63, sxx + x * x, syy + y * y,
                        szz + z * z)

            UNROLL = 8
            def k_body(_, carry):
                kv, sxx, syy, szz = carry
                for _ in range(UNROLL):
                    kv, sxx, syy, szz = point_step(kv, sxx, syy, szz)
                return kv, sxx, syy, szz

            _, sxx, syy, szz = lax.fori_loop(
                0, K // UNROLL, k_body, (iota, zeros, zeros, zeros))

            tot = [zeros, zeros, zeros]
            for o in range(8):
                cnt = accn[pl.ds(o * 16, 16)]
                inv = ones / jnp.maximum(cnt, ones)
                for c, ref in enumerate((accx, accy, accz)):
                    v = ref[pl.ds(o * 16, 16)]
                    tot[c] = tot[c] + v
                    col = rb + jnp.full((16,), 6 + o * 3 + c, jnp.int32)
                    plsc.store_scatter(ob, [zi, col], v * inv)
            sx, sy, sz = tot

            inv_k = jnp.full((16,), 1.0 / K, jnp.float32)
            inv_km1 = jnp.full((16,), 1.0 / (K - 1), jnp.float32)
            tiny = jnp.full((16,), 1e-30, jnp.float32)
            for c, (s1, s2) in enumerate(((sx, sxx), (sy, syy), (sz, szz))):
                var = (s2 - s1 * (s1 * inv_k)) * inv_km1
                var = jnp.maximum(var, zeros)
                std = var * _rsqrt(jnp.maximum(var, tiny))
                plsc.store_scatter(
                    ob, [zi, rb + jnp.full((16,), c, jnp.int32)], std)

            for c in range(3):
                v = plsc.load_gather(
                    cb, [zi, rb + jnp.full((16,), c, jnp.int32)])
                plsc.store_scatter(
                    ob, [zi, rb + jnp.full((16,), 3 + c, jnp.int32)], v)
            return 0

        lax.fori_loop(0, G // 16, sub_body, 0)
        pltpu.sync_copy(ob, out_hbm.at[b, pl.ds(nrow, G), :])
        return 0

    lax.fori_loop(0, NGROUPS, group_body, 0)


@jax.jit
def kernel(group_xyz, new_xyz):
    mesh = plsc.VectorSubcoreMesh(core_axis_name="c", subcore_axis_name="s")
    run = pl.kernel(
        _sc_body,
        out_type=jax.ShapeDtypeStruct((B, N, F), jnp.float32),
        mesh=mesh,
        compiler_params=pltpu.CompilerParams(needs_layout_passes=False),
        scratch_types=[
            pltpu.VMEM((G, K), jnp.float32),
            pltpu.VMEM((G, K), jnp.float32),
            pltpu.VMEM((G, K), jnp.float32),
            pltpu.VMEM((G, 3), jnp.float32),
            pltpu.VMEM((G, F), jnp.float32),
            pltpu.VMEM((128,), jnp.float32),
            pltpu.VMEM((128,), jnp.float32),
            pltpu.VMEM((128,), jnp.float32),
            pltpu.VMEM((128,), jnp.float32),
            pltpu.SemaphoreType.DMA,
        ],
    )
    return run(group_xyz, new_xyz)


# speedup vs baseline: 55.6111x; 1.0119x over previous
import jax
import jax.numpy as jnp
from jax import lax
from jax.experimental import pallas as pl
from jax.experimental.pallas import tpu as pltpu
from jax.experimental.pallas import tpu_sc as plsc

B, C, N, K = 16, 3, 4096, 64
R = B * N
NW = 32
ROWS_PER_W = R // NW
G = 128
NGROUPS = ROWS_PER_W // G
F = 30


def _rsqrt(v):
    i = plsc.bitcast(v, jnp.int32)
    i = jnp.full((16,), 0x5F3759DF, jnp.int32) - lax.shift_right_arithmetic(
        i, jnp.full((16,), 1, jnp.int32))
    y = plsc.bitcast(i, jnp.float32)
    half = jnp.full((16,), 0.5, jnp.float32)
    three_half = jnp.full((16,), 1.5, jnp.float32)
    for _ in range(3):
        y = y * (three_half - half * v * y * y)
    return y


def _sc_body(gx_hbm, nx_hbm, out_hbm, xyzb, cb, ob, accx, accy, accz,
             accn, sem, osem):
    cid = lax.axis_index("c")
    sid = lax.axis_index("s")
    wid = sid * 2 + cid
    b = wid // 2
    n0 = (wid % 2) * ROWS_PER_W
    iota = lax.iota(jnp.int32, 16)
    ones = jnp.full((16,), 1.0, jnp.float32)
    zeros = jnp.zeros((16,), jnp.float32)
    abase = iota + jnp.full((16,), 112, jnp.int32)
    c25 = jnp.full((16,), 25, jnp.int32)
    c26 = jnp.full((16,), 26, jnp.int32)
    c27 = jnp.full((16,), 27, jnp.int32)
    m64 = jnp.full((16,), 64, jnp.int32)
    m32 = jnp.full((16,), 32, jnp.int32)
    m16 = jnp.full((16,), 16, jnp.int32)
    m63 = jnp.full((16,), 63, jnp.int32)
    i1 = jnp.full((16,), 1, jnp.int32)

    def group_body(g, _):
        nrow = n0 + g * G
        h0 = pltpu.async_copy(gx_hbm.at[b, :, pl.ds(nrow, G), :], xyzb, sem)
        h1 = pltpu.async_copy(nx_hbm.at[b, pl.ds(nrow, G), :], cb, sem)
        h0.wait()
        h1.wait()
        @pl.when(g > 0)
        def _():
            pltpu.make_async_copy(
                ob, out_hbm.at[b, pl.ds(nrow, G), :], osem).wait()

        def sub_body(s, _):
            r = s * 16 + iota
            rb = r * jnp.full((16,), 128, jnp.int32)
            zi = jnp.zeros((16,), jnp.int32)
            poy = jnp.full((16,), G * 128, jnp.int32)
            poz = jnp.full((16,), 2 * G * 128, jnp.int32)

            for ref in (accx, accy, accz, accn):
                for i in range(8):
                    ref[pl.ds(i * 16, 16)] = zeros

            def point_step(kv, sxx, syy, szz):
                idx = rb + kv
                x = plsc.load_gather(xyzb, [zi, zi, idx])
                y = plsc.load_gather(xyzb, [zi, zi, idx + poy])
                z = plsc.load_gather(xyzb, [zi, zi, idx + poz])
                sx = lax.shift_right_logical(
                    plsc.bitcast(x, jnp.int32), c25) & m64
                sy = lax.shift_right_logical(
                    plsc.bitcast(y, jnp.int32), c26) & m32
                sz = lax.shift_right_logical(
                    plsc.bitcast(z, jnp.int32), c27) & m16
                a = abase - (sx + sy + sz)
                plsc.addupdate_scatter(accx, [a], x)
                plsc.addupdate_scatter(accy, [a], y)
                plsc.addupdate_scatter(accz, [a], z)
                plsc.addupdate_scatter(accn, [a], ones)
                return ((kv + i1) & m63, sxx + x * x, syy + y * y,
                        szz + z * z)

            UNROLL = 8
            def k_body(_, carry):
                kv, sxx, syy, szz = carry
                for _ in range(UNROLL):
                    kv, sxx, syy, szz = point_step(kv, sxx, syy, szz)
                return kv, sxx, syy, szz

            _, sxx, syy, szz = lax.fori_loop(
                0, K // UNROLL, k_body, (iota, zeros, zeros, zeros))

            tot = [zeros, zeros, zeros]
            for o in range(8):
                cnt = accn[pl.ds(o * 16, 16)]
                inv = ones / jnp.maximum(cnt, ones)
                for c, ref in enumerate((accx, accy, accz)):
                    v = ref[pl.ds(o * 16, 16)]
                    tot[c] = tot[c] + v
                    col = rb + jnp.full((16,), 6 + o * 3 + c, jnp.int32)
                    plsc.store_scatter(ob, [zi, col], v * inv)
            sx, sy, sz = tot

            inv_k = jnp.full((16,), 1.0 / K, jnp.float32)
            inv_km1 = jnp.full((16,), 1.0 / (K - 1), jnp.float32)
            tiny = jnp.full((16,), 1e-30, jnp.float32)
            for c, (s1, s2) in enumerate(((sx, sxx), (sy, syy), (sz, szz))):
                var = (s2 - s1 * (s1 * inv_k)) * inv_km1
                var = jnp.maximum(var, zeros)
                std = var * _rsqrt(jnp.maximum(var, tiny))
                plsc.store_scatter(
                    ob, [zi, rb + jnp.full((16,), c, jnp.int32)], std)

            for c in range(3):
                v = plsc.load_gather(
                    cb, [zi, rb + jnp.full((16,), c, jnp.int32)])
                plsc.store_scatter(
                    ob, [zi, rb + jnp.full((16,), 3 + c, jnp.int32)], v)
            return 0

        lax.fori_loop(0, G // 16, sub_body, 0)
        pltpu.async_copy(ob, out_hbm.at[b, pl.ds(nrow, G), :], osem)
        return 0

    lax.fori_loop(0, NGROUPS, group_body, 0)
    pltpu.make_async_copy(
        ob, out_hbm.at[b, pl.ds(n0, G), :], osem).wait()


@jax.jit
def kernel(group_xyz, new_xyz):
    mesh = plsc.VectorSubcoreMesh(core_axis_name="c", subcore_axis_name="s")
    run = pl.kernel(
        _sc_body,
        out_type=jax.ShapeDtypeStruct((B, N, F), jnp.float32),
        mesh=mesh,
        compiler_params=pltpu.CompilerParams(needs_layout_passes=False),
        scratch_types=[
            pltpu.VMEM((C, G, K), jnp.float32),
            pltpu.VMEM((G, 3), jnp.float32),
            pltpu.VMEM((G, F), jnp.float32),
            pltpu.VMEM((128,), jnp.float32),
            pltpu.VMEM((128,), jnp.float32),
            pltpu.VMEM((128,), jnp.float32),
            pltpu.VMEM((128,), jnp.float32),
            pltpu.SemaphoreType.DMA,
            pltpu.SemaphoreType.DMA,
        ],
    )
    return run(group_xyz, new_xyz)
